# Initial kernel scaffold; baseline (speedup 1.0000x reference)
import jax
import jax.numpy as jnp
from jax.experimental import pallas as pl


def kernel(x, edge_index, W1, b1, W2, b2, g0, bt0, g1, bt1):
    raise NotImplementedError("write your pallas kernel here")


# speedup vs baseline: 18.6028x; 18.6028x over previous
import functools

import jax
import jax.numpy as jnp
from jax import lax
from jax.experimental import pallas as pl
from jax.experimental.pallas import tpu as pltpu
from jax.experimental.pallas import tpu_sc as plsc

N = 10000
D = 128
E = 320000

NC = 2
NS = 16
NW = NC * NS

CHUNK = 128
ETOT = E + N
KPW = -(-ETOT // (NW * CHUNK))
EPW = KPW * CHUNK
EPAD = NW * EPW

NPAD = 10240
RPT = NPAD // NS

@functools.lru_cache(maxsize=1)
def _mesh():
    return plsc.VectorSubcoreMesh(core_axis_name="c", subcore_axis_name="s")


def _deg_body(dst_hbm, ones_hbm, zeros_hbm, degp_hbm, idx_v, ones_v, acc, sem):
    c = lax.axis_index("c")
    s = lax.axis_index("s")
    wid = s * NC + c
    pltpu.sync_copy(dst_hbm.at[wid], idx_v)
    pltpu.sync_copy(ones_hbm, ones_v)

    pltpu.sync_copy(zeros_hbm, acc.at[pl.ds(s * RPT, RPT)])
    plsc.subcore_barrier()

    @pl.loop(0, KPW)
    def _(j):
        pltpu.sync_copy(ones_v, acc.at[idx_v.at[j]], add=True)

    plsc.subcore_barrier()
    pltpu.sync_copy(acc.at[pl.ds(s * RPT, RPT)],
                    degp_hbm.at[c, pl.ds(s * RPT, RPT)])


def _deg_kernel(dst_w, ones16, zeros16):
    k = pl.kernel(
        _deg_body,
        out_type=jax.ShapeDtypeStruct((NC, NPAD, 16), jnp.float32),
        mesh=_mesh(),
        scratch_types=[
            pltpu.VMEM((KPW, CHUNK), jnp.int32),
            pltpu.VMEM((CHUNK, 16), jnp.float32),
            pltpu.VMEM_SHARED((NPAD, 16), jnp.float32),
            pltpu.SemaphoreType.DMA,
        ],
    )
    return k(dst_w, ones16, zeros16)


def _agg_body(hp_hbm, src_hbm, dst_hbm, zeros_hbm, aggp_hbm,
              srcv, dstv, rows_v, acc, sem):
    c = lax.axis_index("c")
    s = lax.axis_index("s")
    wid = s * NC + c
    pltpu.sync_copy(src_hbm.at[wid], srcv)
    pltpu.sync_copy(dst_hbm.at[wid], dstv)

    pltpu.sync_copy(zeros_hbm, acc.at[pl.ds(s * RPT, RPT)])
    plsc.subcore_barrier()

    @pl.loop(0, KPW)
    def _(j):
        pltpu.async_copy(hp_hbm.at[srcv.at[j]], rows_v, sem).wait()
        pltpu.sync_copy(rows_v, acc.at[dstv.at[j]], add=True)

    plsc.subcore_barrier()
    pltpu.sync_copy(acc.at[pl.ds(s * RPT, RPT)],
                    aggp_hbm.at[c, pl.ds(s * RPT, RPT)])


def _agg_kernel(hp, src_w, dst_w, zeros128):
    k = pl.kernel(
        _agg_body,
        out_type=jax.ShapeDtypeStruct((NC, NPAD, D), jnp.float32),
        mesh=_mesh(),
        scratch_types=[
            pltpu.VMEM((KPW, CHUNK), jnp.int32),
            pltpu.VMEM((KPW, CHUNK), jnp.int32),
            pltpu.VMEM((CHUNK, D), jnp.float32),
            pltpu.VMEM_SHARED((NPAD, D), jnp.float32),
            pltpu.SemaphoreType.DMA,
        ],
    )
    return k(hp, src_w, dst_w, zeros128)


BR = 1000
GR = N // BR


def _dinv_of(degp_blk):
    deg = degp_blk[0] + degp_blk[1]
    return lax.rsqrt(deg[:, 0:1])


def _mm1_body(x_ref, w1_ref, degp_ref, out_ref):
    dinv = _dinv_of(degp_ref[...])
    h = jnp.dot(x_ref[...], w1_ref[...], preferred_element_type=jnp.float32)
    out_ref[...] = h * dinv


def _mid_body(aggp_ref, h1p_ref, x_ref, degp_ref, w2_ref, a0_ref, c0_ref,
              out_ref):
    dinv = _dinv_of(degp_ref[...])
    agg = aggp_ref[0] + aggp_ref[1]
    y = (agg * dinv) * a0_ref[0:1, :] + c0_ref[0:1, :]
    y = jnp.where(y > 0, y, 0.1 * y) + x_ref[...]
    h2 = jnp.dot(y, w2_ref[...], preferred_element_type=jnp.float32)
    out_ref[...] = h2 * dinv


def _final_body(aggp_ref, h2p_ref, x_ref, degp_ref, a1_ref, c1_ref, out_ref):
    dinv = _dinv_of(degp_ref[...])
    agg = aggp_ref[0] + aggp_ref[1]
    y = (agg * dinv) * a1_ref[0:1, :] + c1_ref[0:1, :]
    y = jnp.where(y > 0, y, 0.1 * y) + x_ref[...]
    mean = jnp.mean(y, axis=-1, keepdims=True)
    cen = y - mean
    var = jnp.mean(cen * cen, axis=-1, keepdims=True)
    out_ref[...] = cen * lax.rsqrt(var + 1e-5)


def _row_spec(shape_tail):
    return pl.BlockSpec((BR,) + shape_tail, lambda i: (i,) + (0,) * len(shape_tail))


_full128 = pl.BlockSpec((D, D), lambda i: (0, 0))
_vec_spec = pl.BlockSpec((8, D), lambda i: (0, 0))
_degp_spec = pl.BlockSpec((NC, BR, 16), lambda i: (0, i, 0))
_aggp_spec = pl.BlockSpec((NC, BR, D), lambda i: (0, i, 0))
_out_sds = jax.ShapeDtypeStruct((N, D), jnp.float32)


def _mm1(x, W1, degp):
    return pl.pallas_call(
        _mm1_body,
        grid=(GR,),
        in_specs=[_row_spec((D,)), _full128, _degp_spec],
        out_specs=_row_spec((D,)),
        out_shape=_out_sds,
    )(x, W1, degp)


def _mid(aggp, h1p, x, degp, W2, a0, c0):
    return pl.pallas_call(
        _mid_body,
        grid=(GR,),
        in_specs=[_aggp_spec, _row_spec((D,)), _row_spec((D,)), _degp_spec,
                  _full128, _vec_spec, _vec_spec],
        out_specs=_row_spec((D,)),
        out_shape=_out_sds,
    )(aggp, h1p, x, degp, W2, a0, c0)


def _final(aggp, h2p, x, degp, a1, c1):
    return pl.pallas_call(
        _final_body,
        grid=(GR,),
        in_specs=[_aggp_spec, _row_spec((D,)), _row_spec((D,)), _degp_spec,
                  _vec_spec, _vec_spec],
        out_specs=_row_spec((D,)),
        out_shape=_out_sds,
    )(aggp, h2p, x, degp, a1, c1)


def kernel(x, edge_index, W1, b1, W2, b2, g0, bt0, g1, bt1):
    loops = jnp.arange(N, dtype=jnp.int32)
    npad_e = EPAD - ETOT
    pad_src = jnp.arange(npad_e, dtype=jnp.int32) % N
    pad_dst = N + jnp.arange(npad_e, dtype=jnp.int32) % (NPAD - N)
    src_w = jnp.concatenate([edge_index[0], loops, pad_src]).reshape(NW, KPW, CHUNK)
    dst_w = jnp.concatenate([edge_index[1], loops, pad_dst]).reshape(NW, KPW, CHUNK)

    ones16 = jnp.ones((CHUNK, 16), jnp.float32)
    zeros16 = jnp.zeros((RPT, 16), jnp.float32)
    zeros128 = jnp.zeros((RPT, D), jnp.float32)

    sbn = 1.0 / jnp.sqrt(jnp.float32(1.0 + 1e-5))
    a0 = jnp.broadcast_to(g0 * sbn, (8, D))
    c0 = jnp.broadcast_to(b1 * g0 * sbn + bt0, (8, D))
    a1 = jnp.broadcast_to(g1 * sbn, (8, D))
    c1 = jnp.broadcast_to(b2 * g1 * sbn + bt1, (8, D))

    degp_full = _deg_kernel(dst_w, ones16, zeros16)
    degp = degp_full[:, :N, :]

    h1p = _mm1(x, W1, degp)
    agg1 = _agg_kernel(h1p, src_w, dst_w, zeros128)[:, :N, :]
    h2p = _mid(agg1, h1p, x, degp, W2, a0, c0)
    agg2 = _agg_kernel(h2p, src_w, dst_w, zeros128)[:, :N, :]
    return _final(agg2, h2p, x, degp, a1, c1)


# speedup vs baseline: 24.7349x; 1.3296x over previous
import functools

import jax
import jax.numpy as jnp
from jax import lax
from jax.experimental import pallas as pl
from jax.experimental.pallas import tpu as pltpu
from jax.experimental.pallas import tpu_sc as plsc

N = 10000
D = 128
E = 320000

NC = 2
NS = 16
NW = NC * NS

CHUNK = 128
ETOT = E + N
KPW = 82
EPW = KPW * CHUNK
EPAD = NW * EPW

NPAD = 10240
RPT = NPAD // NS

@functools.lru_cache(maxsize=1)
def _mesh():
    return plsc.VectorSubcoreMesh(core_axis_name="c", subcore_axis_name="s")


def _unpack_chunk(pkv, j, idxs, row_s, row_d):
    @pl.loop(0, CHUNK // 16)
    def _(k):
        p = pkv[j, pl.ds(k * 16, 16)]
        idxs[row_s, pl.ds(k * 16, 16)] = lax.bitwise_and(p, 0xFFFF)
        idxs[row_d, pl.ds(k * 16, 16)] = lax.shift_right_logical(p, 16)


def _deg_body(pk_hbm, ones_hbm, zeros_hbm, degp_hbm, pkv, idxs, ones_v, acc,
              sem):
    c = lax.axis_index("c")
    s = lax.axis_index("s")
    wid = s * NC + c
    pltpu.sync_copy(pk_hbm.at[wid], pkv)
    pltpu.sync_copy(ones_hbm, ones_v)

    pltpu.sync_copy(zeros_hbm, acc.at[pl.ds(s * RPT, RPT)])
    plsc.subcore_barrier()

    @pl.loop(0, KPW)
    def _(j):
        _unpack_chunk(pkv, j, idxs, 0, 1)
        pltpu.sync_copy(ones_v, acc.at[idxs.at[1]], add=True)

    plsc.subcore_barrier()
    pltpu.sync_copy(acc.at[pl.ds(s * RPT, RPT)],
                    degp_hbm.at[c, pl.ds(s * RPT, RPT)])


def _deg_kernel(pk_w, ones16, zeros16):
    k = pl.kernel(
        _deg_body,
        out_type=jax.ShapeDtypeStruct((NC, NPAD, 16), jnp.float32),
        mesh=_mesh(),
        scratch_types=[
            pltpu.VMEM((KPW, CHUNK), jnp.int32),
            pltpu.VMEM((2, CHUNK), jnp.int32),
            pltpu.VMEM((CHUNK, 16), jnp.float32),
            pltpu.VMEM_SHARED((NPAD, 16), jnp.float32),
            pltpu.SemaphoreType.DMA,
        ],
    )
    return k(pk_w, ones16, zeros16)


def _agg_body(hp_hbm, pk_hbm, zeros_hbm, aggp_hbm,
              pkv, idxs, rows_a, rows_b, acc, sem_a, sem_b):
    c = lax.axis_index("c")
    s = lax.axis_index("s")
    wid = s * NC + c
    pltpu.sync_copy(pk_hbm.at[wid], pkv)

    pltpu.sync_copy(zeros_hbm, acc.at[pl.ds(s * RPT, RPT)])
    plsc.subcore_barrier()

    _unpack_chunk(pkv, 0, idxs, 0, 1)
    pltpu.async_copy(hp_hbm.at[idxs.at[0]], rows_a, sem_a)

    @pl.loop(0, KPW // 2)
    def _(i):
        j = i * 2
        _unpack_chunk(pkv, j + 1, idxs, 2, 3)
        pltpu.make_async_copy(hp_hbm.at[idxs.at[0]], rows_a, sem_a).wait()
        pltpu.async_copy(hp_hbm.at[idxs.at[2]], rows_b, sem_b)
        pltpu.sync_copy(rows_a, acc.at[idxs.at[1]], add=True)

        @pl.when(i < KPW // 2 - 1)
        def _():
            _unpack_chunk(pkv, j + 2, idxs, 0, 1)
            pltpu.async_copy(hp_hbm.at[idxs.at[0]], rows_a, sem_a)

        pltpu.make_async_copy(hp_hbm.at[idxs.at[2]], rows_b, sem_b).wait()
        pltpu.sync_copy(rows_b, acc.at[idxs.at[3]], add=True)

    plsc.subcore_barrier()
    pltpu.sync_copy(acc.at[pl.ds(s * RPT, RPT)],
                    aggp_hbm.at[c, pl.ds(s * RPT, RPT)])


def _agg_kernel(hp, pk_w, zeros128):
    k = pl.kernel(
        _agg_body,
        out_type=jax.ShapeDtypeStruct((NC, NPAD, D), jnp.float32),
        mesh=_mesh(),
        scratch_types=[
            pltpu.VMEM((KPW, CHUNK), jnp.int32),
            pltpu.VMEM((4, CHUNK), jnp.int32),
            pltpu.VMEM((CHUNK, D), jnp.float32),
            pltpu.VMEM((CHUNK, D), jnp.float32),
            pltpu.VMEM_SHARED((NPAD, D), jnp.float32),
            pltpu.SemaphoreType.DMA,
            pltpu.SemaphoreType.DMA,
        ],
    )
    return k(hp, pk_w, zeros128)


BR = 1000
GR = N // BR


def _dinv_of(degp_blk):
    deg = degp_blk[0] + degp_blk[1]
    return lax.rsqrt(deg[:, 0:1])


def _mm1_body(x_ref, w1_ref, out_ref):
    out_ref[...] = jnp.dot(x_ref[...], w1_ref[...],
                           preferred_element_type=jnp.float32)


def _scale_body(h_ref, degp_ref, out_ref):
    out_ref[...] = h_ref[...] * _dinv_of(degp_ref[...])


def _mid_body(aggp_ref, h1p_ref, x_ref, degp_ref, w2_ref, a0_ref, c0_ref,
              out_ref):
    dinv = _dinv_of(degp_ref[...])
    agg = aggp_ref[0] + aggp_ref[1]
    y = (agg * dinv) * a0_ref[0:1, :] + c0_ref[0:1, :]
    y = jnp.where(y > 0, y, 0.1 * y) + x_ref[...]
    h2 = jnp.dot(y, w2_ref[...], preferred_element_type=jnp.float32)
    out_ref[...] = h2 * dinv


def _final_body(aggp_ref, h2p_ref, x_ref, degp_ref, a1_ref, c1_ref, out_ref):
    dinv = _dinv_of(degp_ref[...])
    agg = aggp_ref[0] + aggp_ref[1]
    y = (agg * dinv) * a1_ref[0:1, :] + c1_ref[0:1, :]
    y = jnp.where(y > 0, y, 0.1 * y) + x_ref[...]
    mean = jnp.mean(y, axis=-1, keepdims=True)
    cen = y - mean
    var = jnp.mean(cen * cen, axis=-1, keepdims=True)
    out_ref[...] = cen * lax.rsqrt(var + 1e-5)


def _row_spec(shape_tail):
    return pl.BlockSpec((BR,) + shape_tail, lambda i: (i,) + (0,) * len(shape_tail))


_full128 = pl.BlockSpec((D, D), lambda i: (0, 0))
_vec_spec = pl.BlockSpec((8, D), lambda i: (0, 0))
_degp_spec = pl.BlockSpec((NC, BR, 16), lambda i: (0, i, 0))
_aggp_spec = pl.BlockSpec((NC, BR, D), lambda i: (0, i, 0))
_out_sds = jax.ShapeDtypeStruct((N, D), jnp.float32)


def _mm1(x, W1):
    return pl.pallas_call(
        _mm1_body,
        grid=(GR,),
        in_specs=[_row_spec((D,)), _full128],
        out_specs=_row_spec((D,)),
        out_shape=_out_sds,
    )(x, W1)


def _scale(h, degp):
    return pl.pallas_call(
        _scale_body,
        grid=(GR,),
        in_specs=[_row_spec((D,)), _degp_spec],
        out_specs=_row_spec((D,)),
        out_shape=_out_sds,
    )(h, degp)


def _mid(aggp, h1p, x, degp, W2, a0, c0):
    return pl.pallas_call(
        _mid_body,
        grid=(GR,),
        in_specs=[_aggp_spec, _row_spec((D,)), _row_spec((D,)), _degp_spec,
                  _full128, _vec_spec, _vec_spec],
        out_specs=_row_spec((D,)),
        out_shape=_out_sds,
    )(aggp, h1p, x, degp, W2, a0, c0)


def _final(aggp, h2p, x, degp, a1, c1):
    return pl.pallas_call(
        _final_body,
        grid=(GR,),
        in_specs=[_aggp_spec, _row_spec((D,)), _row_spec((D,)), _degp_spec,
                  _vec_spec, _vec_spec],
        out_specs=_row_spec((D,)),
        out_shape=_out_sds,
    )(aggp, h2p, x, degp, a1, c1)


def kernel(x, edge_index, W1, b1, W2, b2, g0, bt0, g1, bt1):
    loops = jnp.arange(N, dtype=jnp.int32)
    npad_e = EPAD - ETOT
    pad_src = jnp.arange(npad_e, dtype=jnp.int32) % N
    pad_dst = N + jnp.arange(npad_e, dtype=jnp.int32) % (NPAD - N)
    src_all = jnp.concatenate([edge_index[0], loops, pad_src])
    dst_all = jnp.concatenate([edge_index[1], loops, pad_dst])
    pk_w = (src_all | (dst_all << 16)).reshape(NW, KPW, CHUNK)

    ones16 = jnp.ones((CHUNK, 16), jnp.float32)
    zeros16 = jnp.zeros((RPT, 16), jnp.float32)
    zeros128 = jnp.zeros((RPT, D), jnp.float32)

    sbn = 1.0 / jnp.sqrt(jnp.float32(1.0 + 1e-5))
    a0 = jnp.broadcast_to(g0 * sbn, (8, D))
    c0 = jnp.broadcast_to(b1 * g0 * sbn + bt0, (8, D))
    a1 = jnp.broadcast_to(g1 * sbn, (8, D))
    c1 = jnp.broadcast_to(b2 * g1 * sbn + bt1, (8, D))

    degp_full = _deg_kernel(pk_w, ones16, zeros16)
    degp = degp_full[:, :N, :]

    h1 = _mm1(x, W1)
    h1p = _scale(h1, degp)
    agg1 = _agg_kernel(h1p, pk_w, zeros128)[:, :N, :]
    h2p = _mid(agg1, h1p, x, degp, W2, a0, c0)
    agg2 = _agg_kernel(h2p, pk_w, zeros128)[:, :N, :]
    return _final(agg2, h2p, x, degp, a1, c1)


# speedup vs baseline: 27.2057x; 1.0999x over previous
import functools

import jax
import jax.numpy as jnp
from jax import lax
from jax.experimental import pallas as pl
from jax.experimental.pallas import tpu as pltpu
from jax.experimental.pallas import tpu_sc as plsc

N = 10000
D = 128
E = 320000

NC = 2
NS = 16
NW = NC * NS

CHUNK = 128
ETOT = E + N
KPW = 82
EPW = KPW * CHUNK
EPAD = NW * EPW

NPAD = 10240
RPT = NPAD // NS

@functools.lru_cache(maxsize=1)
def _mesh():
    return plsc.VectorSubcoreMesh(core_axis_name="c", subcore_axis_name="s")


def _unpack_chunk(pkv, j, idxs, row_s, row_d):
    @pl.loop(0, CHUNK // 16)
    def _(k):
        p = pkv[j, pl.ds(k * 16, 16)]
        idxs[row_s, pl.ds(k * 16, 16)] = lax.bitwise_and(p, 0xFFFF)
        idxs[row_d, pl.ds(k * 16, 16)] = lax.shift_right_logical(p, 16)


def _deg_body(pk_hbm, ones_hbm, zeros_hbm, degp_hbm, pkv, dstv, ones_v, acc,
              semz, sem):
    c = lax.axis_index("c")
    s = lax.axis_index("s")
    wid = s * NC + c
    zcp = pltpu.async_copy(zeros_hbm, acc.at[pl.ds(s * RPT, RPT)], semz)
    pltpu.sync_copy(pk_hbm.at[wid], pkv)
    pltpu.sync_copy(ones_hbm, ones_v)

    @pl.loop(0, KPW)
    def _(j):
        @pl.loop(0, CHUNK // 16)
        def _(k):
            p = pkv[j, pl.ds(k * 16, 16)]
            dstv[j, pl.ds(k * 16, 16)] = lax.shift_right_logical(p, 16)

    zcp.wait()
    plsc.subcore_barrier()

    @pl.loop(0, KPW)
    def _(j):
        pltpu.async_copy(ones_v, acc.at[dstv.at[j]], sem, add=True)

    @pl.loop(0, KPW)
    def _(j):
        pltpu.make_async_copy(ones_v, acc.at[dstv.at[0]], sem).wait()

    plsc.subcore_barrier()
    pltpu.sync_copy(acc.at[pl.ds(s * RPT, RPT)],
                    degp_hbm.at[c, pl.ds(s * RPT, RPT)])


def _deg_kernel(pk_w, ones16, zeros16):
    k = pl.kernel(
        _deg_body,
        out_type=jax.ShapeDtypeStruct((NC, NPAD, 16), jnp.float32),
        mesh=_mesh(),
        scratch_types=[
            pltpu.VMEM((KPW, CHUNK), jnp.int32),
            pltpu.VMEM((KPW, CHUNK), jnp.int32),
            pltpu.VMEM((CHUNK, 16), jnp.float32),
            pltpu.VMEM_SHARED((NPAD, 16), jnp.float32),
            pltpu.SemaphoreType.DMA,
            pltpu.SemaphoreType.DMA,
        ],
    )
    return k(pk_w, ones16, zeros16)


def _agg_body(hp_hbm, pk_hbm, zeros_hbm, aggp_hbm,
              pkv, idxs, rows_a, rows_b, acc, semz, sem_a, sem_b):
    c = lax.axis_index("c")
    s = lax.axis_index("s")
    wid = s * NC + c
    zcp = pltpu.async_copy(zeros_hbm, acc.at[pl.ds(s * RPT, RPT)], semz)
    pltpu.sync_copy(pk_hbm.at[wid], pkv)

    _unpack_chunk(pkv, 0, idxs, 0, 1)
    pltpu.async_copy(hp_hbm.at[idxs.at[0]], rows_a, sem_a)
    zcp.wait()
    plsc.subcore_barrier()

    @pl.loop(0, KPW // 2)
    def _(i):
        j = i * 2
        _unpack_chunk(pkv, j + 1, idxs, 2, 3)
        pltpu.make_async_copy(hp_hbm.at[idxs.at[0]], rows_a, sem_a).wait()
        pltpu.async_copy(hp_hbm.at[idxs.at[2]], rows_b, sem_b)
        pltpu.sync_copy(rows_a, acc.at[idxs.at[1]], add=True)

        @pl.when(i < KPW // 2 - 1)
        def _():
            _unpack_chunk(pkv, j + 2, idxs, 0, 1)
            pltpu.async_copy(hp_hbm.at[idxs.at[0]], rows_a, sem_a)

        pltpu.make_async_copy(hp_hbm.at[idxs.at[2]], rows_b, sem_b).wait()
        pltpu.sync_copy(rows_b, acc.at[idxs.at[3]], add=True)

    plsc.subcore_barrier()
    pltpu.sync_copy(acc.at[pl.ds(s * RPT, RPT)],
                    aggp_hbm.at[c, pl.ds(s * RPT, RPT)])


def _agg_kernel(hp, pk_w, zeros128):
    k = pl.kernel(
        _agg_body,
        out_type=jax.ShapeDtypeStruct((NC, NPAD, D), jnp.float32),
        mesh=_mesh(),
        scratch_types=[
            pltpu.VMEM((KPW, CHUNK), jnp.int32),
            pltpu.VMEM((4, CHUNK), jnp.int32),
            pltpu.VMEM((CHUNK, D), jnp.float32),
            pltpu.VMEM((CHUNK, D), jnp.float32),
            pltpu.VMEM_SHARED((NPAD, D), jnp.float32),
            pltpu.SemaphoreType.DMA,
            pltpu.SemaphoreType.DMA,
            pltpu.SemaphoreType.DMA,
        ],
    )
    return k(hp, pk_w, zeros128)


BR = 1000
GR = N // BR


def _dinv_of(degp_blk):
    deg = degp_blk[0] + degp_blk[1]
    return lax.rsqrt(deg[:, 0:1])


def _mm1_body(x_ref, w1_ref, degp_ref, out_ref):
    dinv = _dinv_of(degp_ref[...])
    h = jnp.dot(x_ref[...], w1_ref[...], preferred_element_type=jnp.float32)
    out_ref[...] = h * dinv


def _mid_body(aggp_ref, h1p_ref, x_ref, degp_ref, w2_ref, a0_ref, c0_ref,
              out_ref):
    dinv = _dinv_of(degp_ref[...])
    agg = aggp_ref[0] + aggp_ref[1]
    y = (agg * dinv) * a0_ref[0:1, :] + c0_ref[0:1, :]
    y = jnp.where(y > 0, y, 0.1 * y) + x_ref[...]
    h2 = jnp.dot(y, w2_ref[...], preferred_element_type=jnp.float32)
    out_ref[...] = h2 * dinv


def _final_body(aggp_ref, h2p_ref, x_ref, degp_ref, a1_ref, c1_ref, out_ref):
    dinv = _dinv_of(degp_ref[...])
    agg = aggp_ref[0] + aggp_ref[1]
    y = (agg * dinv) * a1_ref[0:1, :] + c1_ref[0:1, :]
    y = jnp.where(y > 0, y, 0.1 * y) + x_ref[...]
    mean = jnp.mean(y, axis=-1, keepdims=True)
    cen = y - mean
    var = jnp.mean(cen * cen, axis=-1, keepdims=True)
    out_ref[...] = cen * lax.rsqrt(var + 1e-5)


def _row_spec(shape_tail):
    return pl.BlockSpec((BR,) + shape_tail, lambda i: (i,) + (0,) * len(shape_tail))


_full128 = pl.BlockSpec((D, D), lambda i: (0, 0))
_vec_spec = pl.BlockSpec((8, D), lambda i: (0, 0))
_degp_spec = pl.BlockSpec((NC, BR, 16), lambda i: (0, i, 0))
_aggp_spec = pl.BlockSpec((NC, BR, D), lambda i: (0, i, 0))
_out_sds = jax.ShapeDtypeStruct((N, D), jnp.float32)


def _mm1(x, W1, degp):
    return pl.pallas_call(
        _mm1_body,
        grid=(GR,),
        in_specs=[_row_spec((D,)), _full128, _degp_spec],
        out_specs=_row_spec((D,)),
        out_shape=_out_sds,
    )(x, W1, degp)


def _mid(aggp, h1p, x, degp, W2, a0, c0):
    return pl.pallas_call(
        _mid_body,
        grid=(GR,),
        in_specs=[_aggp_spec, _row_spec((D,)), _row_spec((D,)), _degp_spec,
                  _full128, _vec_spec, _vec_spec],
        out_specs=_row_spec((D,)),
        out_shape=_out_sds,
    )(aggp, h1p, x, degp, W2, a0, c0)


def _final(aggp, h2p, x, degp, a1, c1):
    return pl.pallas_call(
        _final_body,
        grid=(GR,),
        in_specs=[_aggp_spec, _row_spec((D,)), _row_spec((D,)), _degp_spec,
                  _vec_spec, _vec_spec],
        out_specs=_row_spec((D,)),
        out_shape=_out_sds,
    )(aggp, h2p, x, degp, a1, c1)


def kernel(x, edge_index, W1, b1, W2, b2, g0, bt0, g1, bt1):
    loops = jnp.arange(N, dtype=jnp.int32)
    npad_e = EPAD - ETOT
    pad_src = jnp.arange(npad_e, dtype=jnp.int32) % N
    pad_dst = N + jnp.arange(npad_e, dtype=jnp.int32) % (NPAD - N)
    src_all = jnp.concatenate([edge_index[0], loops, pad_src])
    dst_all = jnp.concatenate([edge_index[1], loops, pad_dst])
    pk_w = (src_all | (dst_all << 16)).reshape(NW, KPW, CHUNK)

    ones16 = jnp.ones((CHUNK, 16), jnp.float32)
    zeros16 = jnp.zeros((RPT, 16), jnp.float32)
    zeros128 = jnp.zeros((RPT, D), jnp.float32)

    sbn = 1.0 / jnp.sqrt(jnp.float32(1.0 + 1e-5))
    a0 = jnp.broadcast_to(g0 * sbn, (8, D))
    c0 = jnp.broadcast_to(b1 * g0 * sbn + bt0, (8, D))
    a1 = jnp.broadcast_to(g1 * sbn, (8, D))
    c1 = jnp.broadcast_to(b2 * g1 * sbn + bt1, (8, D))

    degp = _deg_kernel(pk_w, ones16, zeros16)

    h1p = _mm1(x, W1, degp)
    agg1 = _agg_kernel(h1p, pk_w, zeros128)
    h2p = _mid(agg1, h1p, x, degp, W2, a0, c0)
    agg2 = _agg_kernel(h2p, pk_w, zeros128)
    return _final(agg2, h2p, x, degp, a1, c1)


# speedup vs baseline: 28.4253x; 1.0448x over previous
import functools

import jax
import jax.numpy as jnp
from jax import lax
from jax.experimental import pallas as pl
from jax.experimental.pallas import tpu as pltpu
from jax.experimental.pallas import tpu_sc as plsc

N = 10000
D = 128
E = 320000

NC = 2
NS = 16
NW = NC * NS

CHUNK = 128
ETOT = E
KPW = 80
EPW = KPW * CHUNK
EPAD = NW * EPW

NPAD = 10240
RPT = NPAD // NS

@functools.lru_cache(maxsize=1)
def _mesh():
    return plsc.VectorSubcoreMesh(core_axis_name="c", subcore_axis_name="s")


def _unpack_chunk(pkv, j, idxs, row_s, row_d):
    @pl.loop(0, CHUNK // 16)
    def _(k):
        p = pkv[j, pl.ds(k * 16, 16)]
        idxs[row_s, pl.ds(k * 16, 16)] = lax.bitwise_and(p, 0xFFFF)
        idxs[row_d, pl.ds(k * 16, 16)] = lax.shift_right_logical(p, 16)


def _deg_body(pk_hbm, ones_hbm, zeros_hbm, degp_hbm, pkv, dstv, ones_v, acc,
              semz, sem):
    c = lax.axis_index("c")
    s = lax.axis_index("s")
    wid = s * NC + c
    zcp = pltpu.async_copy(zeros_hbm, acc.at[pl.ds(s * RPT, RPT)], semz)
    pltpu.sync_copy(pk_hbm.at[wid], pkv)
    pltpu.sync_copy(ones_hbm, ones_v)

    @pl.loop(0, KPW)
    def _(j):
        @pl.loop(0, CHUNK // 16)
        def _(k):
            p = pkv[j, pl.ds(k * 16, 16)]
            dstv[j, pl.ds(k * 16, 16)] = lax.shift_right_logical(p, 16)

    zcp.wait()
    plsc.subcore_barrier()

    @pl.loop(0, KPW)
    def _(j):
        pltpu.async_copy(ones_v, acc.at[dstv.at[j]], sem, add=True)

    @pl.loop(0, KPW)
    def _(j):
        pltpu.make_async_copy(ones_v, acc.at[dstv.at[0]], sem).wait()

    plsc.subcore_barrier()
    pltpu.sync_copy(acc.at[pl.ds(s * RPT, RPT)],
                    degp_hbm.at[c, pl.ds(s * RPT, RPT)])


def _deg_kernel(pk_w, ones16, zeros16):
    k = pl.kernel(
        _deg_body,
        out_type=jax.ShapeDtypeStruct((NC, NPAD, 16), jnp.float32),
        mesh=_mesh(),
        scratch_types=[
            pltpu.VMEM((KPW, CHUNK), jnp.int32),
            pltpu.VMEM((KPW, CHUNK), jnp.int32),
            pltpu.VMEM((CHUNK, 16), jnp.float32),
            pltpu.VMEM_SHARED((NPAD, 16), jnp.float32),
            pltpu.SemaphoreType.DMA,
            pltpu.SemaphoreType.DMA,
        ],
    )
    return k(pk_w, ones16, zeros16)


def _agg_body(hp_hbm, pk_hbm, zeros_hbm, aggp_hbm,
              pkv, idxs, rows_a, rows_b, acc, semz, sem_a, sem_b):
    c = lax.axis_index("c")
    s = lax.axis_index("s")
    wid = s * NC + c
    zcp = pltpu.async_copy(zeros_hbm, acc.at[pl.ds(s * RPT, RPT)], semz)
    pltpu.sync_copy(pk_hbm.at[wid], pkv)

    _unpack_chunk(pkv, 0, idxs, 0, 1)
    pltpu.async_copy(hp_hbm.at[idxs.at[0]], rows_a, sem_a)
    zcp.wait()
    plsc.subcore_barrier()

    @pl.loop(0, KPW // 2)
    def _(i):
        j = i * 2
        _unpack_chunk(pkv, j + 1, idxs, 2, 3)
        pltpu.make_async_copy(hp_hbm.at[idxs.at[0]], rows_a, sem_a).wait()
        pltpu.async_copy(hp_hbm.at[idxs.at[2]], rows_b, sem_b)
        pltpu.sync_copy(rows_a, acc.at[idxs.at[1]], add=True)

        @pl.when(i < KPW // 2 - 1)
        def _():
            _unpack_chunk(pkv, j + 2, idxs, 0, 1)
            pltpu.async_copy(hp_hbm.at[idxs.at[0]], rows_a, sem_a)

        pltpu.make_async_copy(hp_hbm.at[idxs.at[2]], rows_b, sem_b).wait()
        pltpu.sync_copy(rows_b, acc.at[idxs.at[3]], add=True)

    plsc.subcore_barrier()
    pltpu.sync_copy(acc.at[pl.ds(s * RPT, RPT)],
                    aggp_hbm.at[c, pl.ds(s * RPT, RPT)])


def _agg_kernel(hp, pk_w, zeros128):
    k = pl.kernel(
        _agg_body,
        out_type=jax.ShapeDtypeStruct((NC, NPAD, D), jnp.float32),
        mesh=_mesh(),
        scratch_types=[
            pltpu.VMEM((KPW, CHUNK), jnp.int32),
            pltpu.VMEM((4, CHUNK), jnp.int32),
            pltpu.VMEM((CHUNK, D), jnp.float32),
            pltpu.VMEM((CHUNK, D), jnp.float32),
            pltpu.VMEM_SHARED((NPAD, D), jnp.float32),
            pltpu.SemaphoreType.DMA,
            pltpu.SemaphoreType.DMA,
            pltpu.SemaphoreType.DMA,
        ],
    )
    return k(hp, pk_w, zeros128)


BR = 1000
GR = N // BR


def _dinv_of(degp_blk):
    deg = degp_blk[0] + degp_blk[1] + 1.0
    return lax.rsqrt(deg[:, 0:1])


def _mm1_body(x_ref, w1_ref, degp_ref, out_ref):
    dinv = _dinv_of(degp_ref[...])
    h = jnp.dot(x_ref[...], w1_ref[...], preferred_element_type=jnp.float32)
    out_ref[...] = h * dinv


def _mid_body(aggp_ref, h1p_ref, x_ref, degp_ref, w2_ref, a0_ref, c0_ref,
              out_ref):
    dinv = _dinv_of(degp_ref[...])
    agg = aggp_ref[0] + aggp_ref[1] + h1p_ref[...]
    y = (agg * dinv) * a0_ref[0:1, :] + c0_ref[0:1, :]
    y = jnp.where(y > 0, y, 0.1 * y) + x_ref[...]
    h2 = jnp.dot(y, w2_ref[...], preferred_element_type=jnp.float32)
    out_ref[...] = h2 * dinv


def _final_body(aggp_ref, h2p_ref, x_ref, degp_ref, a1_ref, c1_ref, out_ref):
    dinv = _dinv_of(degp_ref[...])
    agg = aggp_ref[0] + aggp_ref[1] + h2p_ref[...]
    y = (agg * dinv) * a1_ref[0:1, :] + c1_ref[0:1, :]
    y = jnp.where(y > 0, y, 0.1 * y) + x_ref[...]
    mean = jnp.mean(y, axis=-1, keepdims=True)
    cen = y - mean
    var = jnp.mean(cen * cen, axis=-1, keepdims=True)
    out_ref[...] = cen * lax.rsqrt(var + 1e-5)


def _row_spec(shape_tail):
    return pl.BlockSpec((BR,) + shape_tail, lambda i: (i,) + (0,) * len(shape_tail))


_full128 = pl.BlockSpec((D, D), lambda i: (0, 0))
_vec_spec = pl.BlockSpec((8, D), lambda i: (0, 0))
_degp_spec = pl.BlockSpec((NC, BR, 16), lambda i: (0, i, 0))
_aggp_spec = pl.BlockSpec((NC, BR, D), lambda i: (0, i, 0))
_out_sds = jax.ShapeDtypeStruct((N, D), jnp.float32)


def _mm1(x, W1, degp):
    return pl.pallas_call(
        _mm1_body,
        grid=(GR,),
        in_specs=[_row_spec((D,)), _full128, _degp_spec],
        out_specs=_row_spec((D,)),
        out_shape=_out_sds,
    )(x, W1, degp)


def _mid(aggp, h1p, x, degp, W2, a0, c0):
    return pl.pallas_call(
        _mid_body,
        grid=(GR,),
        in_specs=[_aggp_spec, _row_spec((D,)), _row_spec((D,)), _degp_spec,
                  _full128, _vec_spec, _vec_spec],
        out_specs=_row_spec((D,)),
        out_shape=_out_sds,
    )(aggp, h1p, x, degp, W2, a0, c0)


def _final(aggp, h2p, x, degp, a1, c1):
    return pl.pallas_call(
        _final_body,
        grid=(GR,),
        in_specs=[_aggp_spec, _row_spec((D,)), _row_spec((D,)), _degp_spec,
                  _vec_spec, _vec_spec],
        out_specs=_row_spec((D,)),
        out_shape=_out_sds,
    )(aggp, h2p, x, degp, a1, c1)


def kernel(x, edge_index, W1, b1, W2, b2, g0, bt0, g1, bt1):
    npad_e = EPAD - ETOT
    pad_src = jnp.arange(npad_e, dtype=jnp.int32) % N
    pad_dst = N + jnp.arange(npad_e, dtype=jnp.int32) % (NPAD - N)
    src_all = jnp.concatenate([edge_index[0], pad_src])
    dst_all = jnp.concatenate([edge_index[1], pad_dst])
    pk_w = (src_all | (dst_all << 16)).reshape(NW, KPW, CHUNK)

    ones16 = jnp.ones((CHUNK, 16), jnp.float32)
    zeros16 = jnp.zeros((RPT, 16), jnp.float32)
    zeros128 = jnp.zeros((RPT, D), jnp.float32)

    sbn = 1.0 / jnp.sqrt(jnp.float32(1.0 + 1e-5))
    a0 = jnp.broadcast_to(g0 * sbn, (8, D))
    c0 = jnp.broadcast_to(b1 * g0 * sbn + bt0, (8, D))
    a1 = jnp.broadcast_to(g1 * sbn, (8, D))
    c1 = jnp.broadcast_to(b2 * g1 * sbn + bt1, (8, D))

    degp = _deg_kernel(pk_w, ones16, zeros16)

    h1p = _mm1(x, W1, degp)
    agg1 = _agg_kernel(h1p, pk_w, zeros128)
    h2p = _mid(agg1, h1p, x, degp, W2, a0, c0)
    agg2 = _agg_kernel(h2p, pk_w, zeros128)
    return _final(agg2, h2p, x, degp, a1, c1)
